# baseline (speedup 1.0000x reference)
import jax
import jax.numpy as jnp
from jax.experimental import pallas as pl

_BLK = 1024


def _router_kernel(x_ref, w_ref, o_ref):
    o_ref[...] = jax.lax.dot_general(
        w_ref[...], x_ref[...],
        dimension_numbers=(((1,), (1,)), ((), ())),
        preferred_element_type=jnp.float32,
    )


def kernel(x, weight):
    hidden = weight.shape[1]
    xf = x.reshape(-1, hidden)
    rows = xf.shape[0]
    n_exp = weight.shape[0]
    out_t = pl.pallas_call(
        _router_kernel,
        grid=(rows // _BLK,),
        in_specs=[
            pl.BlockSpec((_BLK, hidden), lambda i: (i, 0)),
            pl.BlockSpec((n_exp, hidden), lambda i: (0, 0)),
        ],
        out_specs=pl.BlockSpec((n_exp, _BLK), lambda i: (0, i)),
        out_shape=jax.ShapeDtypeStruct((n_exp, rows), jnp.float32),
    )(xf, weight)
    return out_t.T
